# Initial kernel scaffold; baseline (speedup 1.0000x reference)
import jax
import jax.numpy as jnp
from jax.experimental import pallas as pl


def kernel(types, emb):
    raise NotImplementedError("write your pallas kernel here")


# speedup vs baseline: 1.4957x; 1.4957x over previous
import functools

import jax
import jax.numpy as jnp
from jax import lax
from jax.experimental import pallas as pl
from jax.experimental.pallas import tpu as pltpu
from jax.experimental.pallas import tpu_sc as plsc

D_MODEL = 1024
T = 8192

_NC = 2
_NS = 16
_NW = _NC * _NS
_BPW = T // _NW
_C = 64
_NCHUNK = _BPW // _C


@functools.partial(
    pl.kernel,
    mesh=plsc.VectorSubcoreMesh(core_axis_name="c", subcore_axis_name="s"),
    out_type=jax.ShapeDtypeStruct((T, D_MODEL), jnp.float32),
    scratch_types=[
        pltpu.VMEM((_BPW,), jnp.int32),
        pltpu.VMEM((_C, D_MODEL), jnp.float32),
        pltpu.SemaphoreType.DMA,
    ],
)
def _gather_rows(types_hbm, emb_hbm, out_hbm, idx_v, rows_v, sem):
    wid = lax.axis_index("s") * _NC + lax.axis_index("c")
    base = wid * _BPW
    pltpu.sync_copy(types_hbm.at[pl.ds(base, _BPW)], idx_v)
    for c in range(_NCHUNK):
        pltpu.async_copy(
            emb_hbm.at[idx_v.at[pl.ds(c * _C, _C)]], rows_v, sem
        ).wait()
        pltpu.sync_copy(rows_v, out_hbm.at[pl.ds(base + c * _C, _C)])


def kernel(types, emb):
    y = _gather_rows(types.astype(jnp.int32), emb)
    return y[None, :, :]


# speedup vs baseline: 1.5224x; 1.0179x over previous
import functools

import jax
import jax.numpy as jnp
from jax import lax
from jax.experimental import pallas as pl
from jax.experimental.pallas import tpu as pltpu
from jax.experimental.pallas import tpu_sc as plsc

D_MODEL = 1024
T = 8192

_NC = 2
_NS = 16
_NW = _NC * _NS
_BPW = T // _NW
_C = 32
_NCHUNK = _BPW // _C


@functools.partial(
    pl.kernel,
    mesh=plsc.VectorSubcoreMesh(core_axis_name="c", subcore_axis_name="s"),
    out_type=jax.ShapeDtypeStruct((T, D_MODEL), jnp.float32),
    scratch_types=[
        pltpu.VMEM((_BPW,), jnp.int32),
        pltpu.VMEM((_C, D_MODEL), jnp.float32),
        pltpu.VMEM((_C, D_MODEL), jnp.float32),
        pltpu.SemaphoreType.DMA,
        pltpu.SemaphoreType.DMA,
        pltpu.SemaphoreType.DMA,
        pltpu.SemaphoreType.DMA,
    ],
)
def _gather_rows(types_hbm, emb_hbm, out_hbm, idx_v, buf0, buf1,
                 g0, g1, w0, w1):
    wid = lax.axis_index("s") * _NC + lax.axis_index("c")
    base = wid * _BPW
    pltpu.sync_copy(types_hbm.at[pl.ds(base, _BPW)], idx_v)
    bufs = (buf0, buf1)
    gsem = (g0, g1)
    wsem = (w0, w1)
    gh = [None] * _NCHUNK
    wh = [None] * _NCHUNK
    gh[0] = pltpu.async_copy(
        emb_hbm.at[idx_v.at[pl.ds(0, _C)]], bufs[0], gsem[0])
    for c in range(_NCHUNK):
        b = c % 2
        if c + 1 < _NCHUNK:
            nb = (c + 1) % 2
            if c >= 1:
                wh[c - 1].wait()
            gh[c + 1] = pltpu.async_copy(
                emb_hbm.at[idx_v.at[pl.ds((c + 1) * _C, _C)]],
                bufs[nb], gsem[nb])
        gh[c].wait()
        wh[c] = pltpu.async_copy(
            bufs[b], out_hbm.at[pl.ds(base + c * _C, _C)], wsem[b])
    wh[_NCHUNK - 2].wait()
    wh[_NCHUNK - 1].wait()


def kernel(types, emb):
    y = _gather_rows(types.astype(jnp.int32), emb)
    return y[None, :, :]
